# baseline (speedup 1.0000x reference)
import functools

import jax
import jax.numpy as jnp
from jax import lax
from jax.experimental import pallas as pl
from jax.experimental.pallas import tpu as pltpu
from jax.experimental.pallas import tpu_sc as plsc

_MAX_IDX = 64
_L = 16
_PBLK = 12


def _make_sc_pool(b, n, d, t):
    info = plsc.get_sparse_core_info()
    nw = info.num_cores * info.num_subcores
    spw = (b * n) // nw
    ngrp = spw // _L
    nvec = d // _L
    mesh = plsc.VectorSubcoreMesh(core_axis_name="c", subcore_axis_name="s")

    @functools.partial(
        pl.kernel,
        mesh=mesh,
        out_type=jax.ShapeDtypeStruct((b * n * d,), jnp.float32),
        scratch_types=[
            pltpu.VMEM((2 * spw,), jnp.int32),
            pltpu.VMEM(((_MAX_IDX + 1) * d,), jnp.float32),
            pltpu.VMEM((spw * d,), jnp.float32),
            pltpu.SemaphoreType.DMA,
            pltpu.SemaphoreType.DMA,
            [pltpu.SemaphoreType.DMA] * ngrp,
        ],
        compiler_params=pltpu.CompilerParams(needs_layout_passes=False),
    )
    def pool(seq_hbm, spans_hbm, out_hbm,
             se_v, p_v, out_v, sem_x, sem_p, sems_o):
        wid = lax.axis_index("s") * info.num_cores + lax.axis_index("c")
        base = wid * spw
        bi = base // n

        cp_x = pltpu.async_copy(
            seq_hbm.at[pl.ds(bi * t * d, _MAX_IDX * d)],
            p_v.at[pl.ds(d, _MAX_IDX * d)], sem_x)
        cp_p = pltpu.async_copy(
            spans_hbm.at[pl.ds(2 * base, 2 * spw)], se_v, sem_p)

        lane = lax.iota(jnp.int32, _L)
        zeros = jnp.zeros((_L,), jnp.int32)
        fzeros = jnp.zeros((_L,), jnp.float32)

        cp_p.wait()
        groups = []
        for g in range(ngrp):
            rows = lane + (g * _L)
            s16 = plsc.load_gather(se_v, [rows * 2])
            e16 = plsc.load_gather(se_v, [rows * 2 + 1])
            inv16 = 1.0 / (e16 - s16 + 1).astype(jnp.float32)
            groups.append((s16 * d, (e16 + 1) * d, inv16, rows * d))

        for c in range(nvec):
            p_v[pl.ds(c * _L, _L)] = fzeros
        cp_x.wait()
        for blk in range(0, nvec, _PBLK):
            nb = min(_PBLK, nvec - blk)

            def prefix_row(i, carry, blk=blk, nb=nb):
                off = i * d + blk * _L
                new = []
                for c in range(nb):
                    sl = pl.ds(off + c * _L, _L)
                    acc = carry[c] + p_v[sl]
                    p_v[sl] = acc
                    new.append(acc)
                return tuple(new)

            lax.fori_loop(1, _MAX_IDX + 1, prefix_row, (fzeros,) * nb)

        outs = []
        for g in range(ngrp):
            s16, e16, inv16, rows = groups[g]

            def col_body(c, carry, s16=s16, e16=e16, inv16=inv16, rows=rows):
                pe = plsc.load_gather(p_v, [e16 + c])
                ps = plsc.load_gather(p_v, [s16 + c])
                plsc.store_scatter(out_v, [rows + c], (pe - ps) * inv16)
                return carry

            lax.fori_loop(0, d, col_body, 0, unroll=8)
            outs.append(pltpu.async_copy(
                out_v.at[pl.ds(g * _L * d, _L * d)],
                out_hbm.at[pl.ds((base + g * _L) * d, _L * d)],
                sems_o[g]))

        for cp in outs:
            cp.wait()

    return pool


def kernel(sequence_tensor, span_indices):
    b, t, d = sequence_tensor.shape
    n = span_indices.shape[1]
    seq_flat = sequence_tensor.reshape(b * t * d)
    spans_flat = span_indices.reshape(b * n * 2).astype(jnp.int32)
    pool = _make_sc_pool(b, n, d, t)
    out = pool(seq_flat, spans_flat)
    return out.reshape(b, n, d)


# speedup vs baseline: 1.8860x; 1.8860x over previous
import functools

import jax
import jax.numpy as jnp
from jax import lax
from jax.experimental import pallas as pl
from jax.experimental.pallas import tpu as pltpu
from jax.experimental.pallas import tpu_sc as plsc

_MAX_IDX = 64
_L = 16
_PBLK = 12


def _make_sc_pool(b, n, d, t):
    info = plsc.get_sparse_core_info()
    nw = info.num_cores * info.num_subcores
    spw = (b * n) // nw
    ngrp = spw // _L
    nvec = d // _L
    mesh = plsc.VectorSubcoreMesh(core_axis_name="c", subcore_axis_name="s")

    @functools.partial(
        pl.kernel,
        mesh=mesh,
        out_type=jax.ShapeDtypeStruct((b * n * d,), jnp.float32),
        scratch_types=[
            pltpu.VMEM((2 * spw,), jnp.int32),
            pltpu.VMEM(((_MAX_IDX + 1) * d,), jnp.float32),
            pltpu.VMEM((spw * d,), jnp.float32),
            pltpu.SemaphoreType.DMA,
            pltpu.SemaphoreType.DMA,
            [pltpu.SemaphoreType.DMA] * ngrp,
        ],
        compiler_params=pltpu.CompilerParams(needs_layout_passes=False),
    )
    def pool(seq_hbm, spans_hbm, out_hbm,
             se_v, p_v, out_v, sem_x, sem_p, sems_o):
        wid = lax.axis_index("s") * info.num_cores + lax.axis_index("c")
        base = wid * spw
        bi = base // n

        cp_x = pltpu.async_copy(
            seq_hbm.at[pl.ds(bi * t * d, _MAX_IDX * d)],
            p_v.at[pl.ds(d, _MAX_IDX * d)], sem_x)
        cp_p = pltpu.async_copy(
            spans_hbm.at[pl.ds(2 * base, 2 * spw)], se_v, sem_p)

        lane = lax.iota(jnp.int32, _L)
        zeros = jnp.zeros((_L,), jnp.int32)
        fzeros = jnp.zeros((_L,), jnp.float32)

        cp_p.wait()
        groups = []
        for g in range(ngrp):
            rows = lane + (g * _L)
            s16 = plsc.load_gather(se_v, [rows * 2])
            e16 = plsc.load_gather(se_v, [rows * 2 + 1])
            inv16 = 1.0 / (e16 - s16 + 1).astype(jnp.float32)
            groups.append((s16 * d, (e16 + 1) * d, inv16, rows * d))

        for c in range(nvec):
            p_v[pl.ds(c * _L, _L)] = fzeros
        cp_x.wait()
        for blk in range(0, nvec, _PBLK):
            nb = min(_PBLK, nvec - blk)

            def prefix_row(i, carry, blk=blk, nb=nb):
                off = i * d + blk * _L
                new = []
                for c in range(nb):
                    sl = pl.ds(off + c * _L, _L)
                    acc = carry[c] + p_v[sl]
                    p_v[sl] = acc
                    new.append(acc)
                return tuple(new)

            lax.fori_loop(1, _MAX_IDX + 1, prefix_row, (fzeros,) * nb)

        outs = []
        for g in range(ngrp):
            s16, e16, inv16, rows = groups[g]

            def col_body(c, carry, s16=s16, e16=e16, inv16=inv16, rows=rows):
                colv = (c & -_L) + ((lane + c) & (_L - 1))
                pe = plsc.load_gather(p_v, [e16 + colv])
                ps = plsc.load_gather(p_v, [s16 + colv])
                plsc.store_scatter(out_v, [rows + colv], (pe - ps) * inv16)
                return carry

            lax.fori_loop(0, d, col_body, 0, unroll=8)
            outs.append(pltpu.async_copy(
                out_v.at[pl.ds(g * _L * d, _L * d)],
                out_hbm.at[pl.ds((base + g * _L) * d, _L * d)],
                sems_o[g]))

        for cp in outs:
            cp.wait()

    return pool


def kernel(sequence_tensor, span_indices):
    b, t, d = sequence_tensor.shape
    n = span_indices.shape[1]
    seq_flat = sequence_tensor.reshape(b * t * d)
    spans_flat = span_indices.reshape(b * n * 2).astype(jnp.int32)
    pool = _make_sc_pool(b, n, d, t)
    out = pool(seq_flat, spans_flat)
    return out.reshape(b, n, d)
